# Initial kernel scaffold; baseline (speedup 1.0000x reference)
import jax
import jax.numpy as jnp
from jax.experimental import pallas as pl


def kernel(x, edge_index, edge_weight, weight, bias):
    raise NotImplementedError("write your pallas kernel here")


# speedup vs baseline: 4.1071x; 4.1071x over previous
import functools

import jax
import jax.numpy as jnp
from jax import lax
from jax.experimental import pallas as pl
from jax.experimental.pallas import tpu as pltpu
from jax.experimental.pallas import tpu_sc as plsc

N = 10000
E = 320000
D = 128
MIN_NORM = 1e-15
MAXNORM = 1.0 - 4e-3


def _artanh(x):
    x = jnp.clip(x, -1.0 + 1e-7, 1.0 - 1e-7)
    return 0.5 * (jnp.log1p(x) - jnp.log1p(-x))


def _nrm(x):
    return jnp.clip(jnp.sqrt(jnp.sum(x * x, axis=-1, keepdims=True)), MIN_NORM, None)


def _proj(x):
    n = _nrm(x)
    return jnp.where(n > MAXNORM, x / n * MAXNORM, x)


def _expmap0(u):
    n = _nrm(u)
    return jnp.tanh(n) * u / n


def _logmap0(p):
    n = _nrm(p)
    return _artanh(n) * p / n


def _mobius_add(x, y):
    x2 = jnp.sum(x * x, axis=-1, keepdims=True)
    y2 = jnp.sum(y * y, axis=-1, keepdims=True)
    xy = jnp.sum(x * y, axis=-1, keepdims=True)
    num = (1.0 + 2.0 * xy + y2) * x + (1.0 - x2) * y
    denom = 1.0 + 2.0 * xy + x2 * y2
    return num / jnp.clip(denom, MIN_NORM, None)


def _pre_body(x_ref, w_ref, b_ref, o_ref):
    x = x_ref[...]
    w = w_ref[...]
    mx = lax.dot_general(x, w, (((1,), (1,)), ((), ())),
                         preferred_element_type=jnp.float32)
    x_norm = _nrm(x)
    mx_norm = _nrm(mx)
    res_c = jnp.tanh(mx_norm / x_norm * _artanh(x_norm)) * mx / mx_norm
    cond = jnp.all(mx == 0, axis=-1, keepdims=True)
    mv = jnp.where(cond, jnp.zeros_like(res_c), res_c)
    res = _proj(mv)
    hyp_bias = _proj(_expmap0(_proj(b_ref[...])))
    res = _proj(_mobius_add(res, hyp_bias))
    o_ref[...] = _logmap0(res)


def _pre_body_exact(x_ref, w_ref, b_ref, o_ref):
    x = x_ref[...]
    w = w_ref[...]
    mx = lax.dot_general(x, w, (((1,), (1,)), ((), ())),
                         preferred_element_type=jnp.float32)
    x_norm = _nrm(x)
    mx_norm = _nrm(mx)
    res_c = jnp.tanh(mx_norm / x_norm * _artanh(x_norm)) * mx / mx_norm
    cond = jnp.all(mx == 0, axis=-1, keepdims=True)
    mv = jnp.where(cond, jnp.zeros_like(res_c), res_c)
    res = _proj(mv)
    hyp_bias = _proj(_expmap0(b_ref[...]))
    res = _proj(_mobius_add(res, hyp_bias))
    o_ref[...] = _logmap0(res)


def _pre_stage(x, weight, bias, block_n=2000):
    grid = N // block_n
    return pl.pallas_call(
        _pre_body_exact,
        grid=(grid,),
        in_specs=[
            pl.BlockSpec((block_n, D), lambda i: (i, 0)),
            pl.BlockSpec((D, D), lambda i: (0, 0)),
            pl.BlockSpec((1, D), lambda i: (0, 0)),
        ],
        out_specs=pl.BlockSpec((block_n, D), lambda i: (i, 0)),
        out_shape=jax.ShapeDtypeStruct((N, D), jnp.float32),
    )(x, weight, bias)


CH = 80
NP = 10240


def _sc_body(xt_hbm, src_hbm, dst_hbm, wgt_hbm, out_hbm,
             isrc, idst, wv, rows, stg, acc, sem):
    c = lax.axis_index("c")
    s = lax.axis_index("s")
    wid = c * 16 + s

    rpt = NP // 16
    spt = rpt // 128

    def _zrow(i, carry):
        for g in range(8):
            stg[i, pl.ds(g * 16, 16)] = jnp.zeros((16,), jnp.float32)
        return carry

    lax.fori_loop(0, 128, _zrow, 0)
    base = s * rpt
    for r in range(spt):
        pltpu.sync_copy(stg, acc.at[pl.ds(base + r * 128, 128)])
    plsc.subcore_barrier()

    ept = E // 32
    eb0 = wid * ept

    def _chunk(k, carry):
        eb = eb0 + k * CH
        pltpu.sync_copy(src_hbm.at[pl.ds(eb, CH)], isrc)
        pltpu.sync_copy(dst_hbm.at[pl.ds(eb, CH)], idst)
        pltpu.sync_copy(wgt_hbm.at[pl.ds(eb, CH)], wv)
        pltpu.async_copy(xt_hbm.at[isrc], rows, sem).wait()

        def _scale(g, cc):
            w16 = wv[pl.ds(g * 16, 16)]
            for j in range(16):
                e = g * 16 + j
                w = w16[j]
                for q in range(8):
                    rows[e, pl.ds(q * 16, 16)] = rows[e, pl.ds(q * 16, 16)] * w
            return cc

        lax.fori_loop(0, CH // 16, _scale, 0)
        pltpu.sync_copy(rows, acc.at[idst], add=True)
        return carry

    lax.fori_loop(0, ept // CH, _chunk, 0)
    plsc.subcore_barrier()

    for r in range(spt):
        off = base + r * 128
        pltpu.sync_copy(acc.at[pl.ds(off, 128)], stg)
        pltpu.sync_copy(stg, out_hbm.at[c, pl.ds(off, 128)])


def _spmm_stage(x_t, src, dst, ew):
    mesh = plsc.VectorSubcoreMesh(core_axis_name="c", subcore_axis_name="s")
    f = pl.kernel(
        _sc_body,
        out_type=jax.ShapeDtypeStruct((2, NP, D), jnp.float32),
        mesh=mesh,
        scratch_types=[
            pltpu.VMEM((CH,), jnp.int32),
            pltpu.VMEM((CH,), jnp.int32),
            pltpu.VMEM((CH,), jnp.float32),
            pltpu.VMEM((CH, D), jnp.float32),
            pltpu.VMEM((128, D), jnp.float32),
            pltpu.VMEM_SHARED((NP, D), jnp.float32),
            pltpu.SemaphoreType.DMA,
        ],
    )
    return f(x_t, src, dst, ew)


def _post_body(p_ref, o_ref):
    s = p_ref[0] + p_ref[1]
    h = _proj(_expmap0(s))
    xt = jax.nn.relu(_logmap0(h))
    o_ref[...] = _proj(_expmap0(xt))


def _post_stage(partials, block_n=2000):
    grid = N // block_n
    return pl.pallas_call(
        _post_body,
        grid=(grid,),
        in_specs=[pl.BlockSpec((2, block_n, D), lambda i: (0, i, 0))],
        out_specs=pl.BlockSpec((block_n, D), lambda i: (i, 0)),
        out_shape=jax.ShapeDtypeStruct((N, D), jnp.float32),
    )(partials)


@jax.jit
def kernel(x, edge_index, edge_weight, weight, bias):
    x_t = _pre_stage(x, weight, bias)
    dst = edge_index[0].astype(jnp.int32)
    src = edge_index[1].astype(jnp.int32)
    partials = _spmm_stage(x_t, src, dst, edge_weight)
    return _post_stage(partials)
